# baseline (speedup 1.0000x reference)
import jax
import jax.numpy as jnp
from jax import lax
from jax.experimental import pallas as pl
from jax.experimental.pallas import tpu as pltpu
from jax.experimental.pallas import tpu_sc as plsc

N_NODES = 10000
N_EDGES = 320000
D_FEAT = 128

NC = 2
NS = 16
NW = NC * NS
EPW = N_EDGES // NW
C = 80
NCHUNK = EPW // C
G = C // 16
ROWS_PER_STAGE = 624


def _body(c_hbm, g_hbm, src_hbm, dst_hbm, out_hbm,
          iu, iv, ub, vb, ob, c_spmem,
          sem_iu, sem_iv, sem_u, sem_v, sem_o):
    wid = lax.axis_index("s") * NC + lax.axis_index("c")
    base = wid * EPW
    lane = jnp.arange(16, dtype=jnp.int32)

    sid = lax.axis_index("s")
    pltpu.sync_copy(c_hbm.at[pl.ds(sid * ROWS_PER_STAGE, ROWS_PER_STAGE)],
                    c_spmem.at[pl.ds(sid * ROWS_PER_STAGE, ROWS_PER_STAGE)])

    @pl.when(sid == 0)
    def _():
        rem = NS * ROWS_PER_STAGE
        pltpu.sync_copy(c_hbm.at[pl.ds(rem, N_NODES - rem)],
                        c_spmem.at[pl.ds(rem, N_NODES - rem)])

    plsc.subcore_barrier()

    def fire_idx(ci, p):
        off = base + ci * C
        pltpu.async_copy(src_hbm.at[pl.ds(off, C)], iu[p], sem_iu[p])
        pltpu.async_copy(dst_hbm.at[pl.ds(off, C)], iv[p], sem_iv[p])

    def wait_idx(p):
        pltpu.make_async_copy(src_hbm.at[pl.ds(0, C)], iu[p], sem_iu[p]).wait()
        pltpu.make_async_copy(dst_hbm.at[pl.ds(0, C)], iv[p], sem_iv[p]).wait()

    def fire_rows(p):
        pltpu.async_copy(c_spmem.at[iu[p]], ub[p], sem_u[p])
        pltpu.async_copy(g_hbm.at[iv[p]], vb[p], sem_v[p])

    def wait_rows(p):
        pltpu.make_async_copy(c_spmem.at[iu[p]], ub[p], sem_u[p]).wait()
        pltpu.make_async_copy(g_hbm.at[iv[p]], vb[p], sem_v[p]).wait()

    def fire_out(ci, p):
        pltpu.async_copy(ob[p], out_hbm.at[pl.ds(base + ci * C, C)], sem_o[p])

    def wait_out(p):
        pltpu.make_async_copy(ob[p], out_hbm.at[pl.ds(base, C)], sem_o[p]).wait()

    def compute(p):
        def group_body(gi, _):
            rows = gi * 16 + lane
            cols0 = [lane + k for k in range(4)]
            accs0 = [jnp.zeros((16,), jnp.float32) for _ in range(4)]

            def d_body(t, carry):
                accs, cols = carry
                accs = list(accs)
                cols = list(cols)
                for q in range(4):
                    for k in range(4):
                        hu = plsc.load_gather(ub[p], [rows, cols[k]])
                        hv = plsc.load_gather(vb[p], [rows, cols[k]])
                        accs[k] = accs[k] + hu * hv
                        cols[k] = (cols[k] + 4) & (D_FEAT - 1)
                return tuple(accs), tuple(cols)

            (a0, a1, a2, a3), _ = plsc.parallel_loop(
                0, D_FEAT // 16, 1, unroll=2,
                carry=(tuple(accs0), tuple(cols0)))(d_body)
            acc = (a0 + a1) + (a2 + a3)
            pred = 1.0 / (1.0 + jnp.exp(-acc))
            ob[p][pl.ds(gi * 16, 16)] = pred
            return 0

        lax.fori_loop(0, G, group_body, 0)

    def step(ci, p, has_next, has_next2, has_prev_out):
        if has_next:
            wait_idx(p ^ 1)
            fire_rows(p ^ 1)
        wait_rows(p)
        if has_next2:
            fire_idx(ci + 2, p)
        if has_prev_out:
            wait_out(p)
        compute(p)
        fire_out(ci, p)

    fire_idx(0, 0)
    fire_idx(1, 1)
    wait_idx(0)
    fire_rows(0)

    step(0, 0, True, True, False)
    step(1, 1, True, True, False)

    def pair_body(i2, _):
        ci0 = i2 * 2
        step(ci0, 0, True, True, True)
        step(ci0 + 1, 1, True, True, True)
        return 0

    lax.fori_loop(1, (NCHUNK - 1) // 2 - 1, pair_body, 0)
    step(NCHUNK - 3, 0, True, True, True)
    step(NCHUNK - 2, 1, True, False, True)
    step(NCHUNK - 1, 0, False, False, True)
    wait_out(1)
    wait_out(0)


@jax.jit
def _decode(c_feat, g_feat, src, dst):
    mesh = plsc.VectorSubcoreMesh(core_axis_name="c", subcore_axis_name="s",
                                  num_cores=NC, num_subcores=NS)
    return pl.kernel(
        _body,
        out_type=jax.ShapeDtypeStruct((N_EDGES,), jnp.float32),
        mesh=mesh,
        compiler_params=pltpu.CompilerParams(needs_layout_passes=False),
        scratch_types=[
            [pltpu.VMEM((C,), jnp.int32) for _ in range(2)],
            [pltpu.VMEM((C,), jnp.int32) for _ in range(2)],
            [pltpu.VMEM((C, D_FEAT), jnp.float32) for _ in range(2)],
            [pltpu.VMEM((C, D_FEAT), jnp.float32) for _ in range(2)],
            [pltpu.VMEM((C,), jnp.float32) for _ in range(2)],
            pltpu.VMEM_SHARED((N_NODES, D_FEAT), jnp.float32),
            [pltpu.SemaphoreType.DMA for _ in range(2)],
            [pltpu.SemaphoreType.DMA for _ in range(2)],
            [pltpu.SemaphoreType.DMA for _ in range(2)],
            [pltpu.SemaphoreType.DMA for _ in range(2)],
            [pltpu.SemaphoreType.DMA for _ in range(2)],
        ],
    )(c_feat, g_feat, src, dst)


def kernel(c_feat, g_feat, edge_index):
    ei = edge_index.astype(jnp.int32)
    out = _decode(c_feat, g_feat, ei[0], ei[1])
    return out.reshape(N_EDGES, 1)


# speedup vs baseline: 1.0322x; 1.0322x over previous
import jax
import jax.numpy as jnp
from jax import lax
from jax.experimental import pallas as pl
from jax.experimental.pallas import tpu as pltpu
from jax.experimental.pallas import tpu_sc as plsc

N_NODES = 10000
N_EDGES = 320000
D_FEAT = 128

NC = 2
NS = 16
NW = NC * NS
EPW = N_EDGES // NW
C = 80
NCHUNK = EPW // C
G = C // 16
ROWS_PER_STAGE = 624


def _body(c_hbm, g_hbm, src_hbm, dst_hbm, out_hbm,
          iu, iv, ub, vb, ob, c_spmem,
          sem_iu, sem_iv, sem_u, sem_v, sem_o):
    wid = lax.axis_index("s") * NC + lax.axis_index("c")
    base = wid * EPW
    lane = jnp.arange(16, dtype=jnp.int32)

    sid = lax.axis_index("s")
    pltpu.sync_copy(c_hbm.at[pl.ds(sid * ROWS_PER_STAGE, ROWS_PER_STAGE)],
                    c_spmem.at[pl.ds(sid * ROWS_PER_STAGE, ROWS_PER_STAGE)])

    @pl.when(sid == 0)
    def _():
        rem = NS * ROWS_PER_STAGE
        pltpu.sync_copy(c_hbm.at[pl.ds(rem, N_NODES - rem)],
                        c_spmem.at[pl.ds(rem, N_NODES - rem)])

    plsc.subcore_barrier()

    def fire_idx(ci, p):
        off = base + ci * C
        pltpu.async_copy(src_hbm.at[pl.ds(off, C)], iu[p], sem_iu[p])
        pltpu.async_copy(dst_hbm.at[pl.ds(off, C)], iv[p], sem_iv[p])

    def wait_idx(p):
        pltpu.make_async_copy(src_hbm.at[pl.ds(0, C)], iu[p], sem_iu[p]).wait()
        pltpu.make_async_copy(dst_hbm.at[pl.ds(0, C)], iv[p], sem_iv[p]).wait()

    def fire_rows(p):
        pltpu.async_copy(c_spmem.at[iu[p]], ub[p], sem_u[p])
        pltpu.async_copy(g_hbm.at[iv[p]], vb[p], sem_v[p])

    def wait_rows(p):
        pltpu.make_async_copy(c_spmem.at[iu[p]], ub[p], sem_u[p]).wait()
        pltpu.make_async_copy(g_hbm.at[iv[p]], vb[p], sem_v[p]).wait()

    def fire_out(ci, p):
        pltpu.async_copy(ob[p], out_hbm.at[pl.ds(base + ci * C, C)], sem_o[p])

    def wait_out(p):
        pltpu.make_async_copy(ob[p], out_hbm.at[pl.ds(base, C)], sem_o[p]).wait()

    def compute(p):
        def group_body(gi, _):
            rows = gi * 16 + lane
            cols0 = [lane + k for k in range(4)]
            accs0 = [jnp.zeros((16,), jnp.float32) for _ in range(4)]

            def d_body(t, carry):
                accs, cols = carry
                accs = list(accs)
                cols = list(cols)
                for q in range(4):
                    for k in range(4):
                        hu = plsc.load_gather(ub[p], [rows, cols[k]])
                        hv = plsc.load_gather(vb[p], [rows, cols[k]])
                        accs[k] = accs[k] + hu * hv
                        cols[k] = (cols[k] + 4) & (D_FEAT - 1)
                return tuple(accs), tuple(cols)

            (a0, a1, a2, a3), _ = lax.fori_loop(
                0, D_FEAT // 16, d_body, (tuple(accs0), tuple(cols0)))
            acc = (a0 + a1) + (a2 + a3)
            pred = 1.0 / (1.0 + jnp.exp(-acc))
            ob[p][pl.ds(gi * 16, 16)] = pred
            return 0

        lax.fori_loop(0, G, group_body, 0)

    def step(ci, p, has_next, has_next2, has_prev_out):
        if has_next:
            wait_idx(p ^ 1)
            fire_rows(p ^ 1)
        wait_rows(p)
        if has_next2:
            fire_idx(ci + 2, p)
        if has_prev_out:
            wait_out(p)
        compute(p)
        fire_out(ci, p)

    fire_idx(0, 0)
    fire_idx(1, 1)
    wait_idx(0)
    fire_rows(0)

    step(0, 0, True, True, False)
    step(1, 1, True, True, False)

    def pair_body(i2, _):
        ci0 = i2 * 2
        step(ci0, 0, True, True, True)
        step(ci0 + 1, 1, True, True, True)
        return 0

    lax.fori_loop(1, (NCHUNK - 1) // 2 - 1, pair_body, 0)
    step(NCHUNK - 3, 0, True, True, True)
    step(NCHUNK - 2, 1, True, False, True)
    step(NCHUNK - 1, 0, False, False, True)
    wait_out(1)
    wait_out(0)


@jax.jit
def _decode(c_feat, g_feat, src, dst):
    mesh = plsc.VectorSubcoreMesh(core_axis_name="c", subcore_axis_name="s",
                                  num_cores=NC, num_subcores=NS)
    return pl.kernel(
        _body,
        out_type=jax.ShapeDtypeStruct((N_EDGES,), jnp.float32),
        mesh=mesh,
        compiler_params=pltpu.CompilerParams(needs_layout_passes=False),
        scratch_types=[
            [pltpu.VMEM((C,), jnp.int32) for _ in range(2)],
            [pltpu.VMEM((C,), jnp.int32) for _ in range(2)],
            [pltpu.VMEM((C, D_FEAT), jnp.float32) for _ in range(2)],
            [pltpu.VMEM((C, D_FEAT), jnp.float32) for _ in range(2)],
            [pltpu.VMEM((C,), jnp.float32) for _ in range(2)],
            pltpu.VMEM_SHARED((N_NODES, D_FEAT), jnp.float32),
            [pltpu.SemaphoreType.DMA for _ in range(2)],
            [pltpu.SemaphoreType.DMA for _ in range(2)],
            [pltpu.SemaphoreType.DMA for _ in range(2)],
            [pltpu.SemaphoreType.DMA for _ in range(2)],
            [pltpu.SemaphoreType.DMA for _ in range(2)],
        ],
    )(c_feat, g_feat, src, dst)


def kernel(c_feat, g_feat, edge_index):
    ei = edge_index.astype(jnp.int32)
    out = _decode(c_feat, g_feat, ei[0], ei[1])
    return out.reshape(N_EDGES, 1)
